# baseline (speedup 1.0000x reference)
import jax
import jax.numpy as jnp
from jax.experimental import pallas as pl
from jax.experimental.pallas import tpu as pltpu

DIM_ = 1024
B_, W_ = 64, 2048
TOPK_K = max(1, int(round(W_ * 0.1)))
BETA = 0.6
TILE_A = 256
DEPTH = 24
N_TILES = (B_ * W_) // TILE_A


ODEPTH = 4


def _matmul_kernel(x_hbm, w_ref, o_hbm, buf, sem, obuf, osem):
    wbf = w_ref[...].astype(jnp.bfloat16)

    def start_copy(i, slot):
        pltpu.make_async_copy(
            x_hbm.at[pl.ds(i * TILE_A, TILE_A), :],
            buf.at[slot], sem.at[slot]).start()

    def out_copy(i, oslot):
        return pltpu.make_async_copy(
            obuf.at[oslot],
            o_hbm.at[pl.ds(i * TILE_A, TILE_A), :], osem.at[oslot])

    for j in range(DEPTH):
        start_copy(j, j)

    def body(i, _):
        slot = jax.lax.rem(i, DEPTH)
        oslot = jax.lax.rem(i, ODEPTH)
        pltpu.make_async_copy(
            x_hbm.at[pl.ds(i * TILE_A, TILE_A), :],
            buf.at[slot], sem.at[slot]).wait()

        @pl.when(i >= ODEPTH)
        def _():
            out_copy(i - ODEPTH, oslot).wait()

        obuf[oslot] = jnp.dot(
            buf[slot].astype(jnp.bfloat16), wbf,
            preferred_element_type=jnp.float32)
        out_copy(i, oslot).start()

        @pl.when(i + DEPTH < N_TILES)
        def _():
            start_copy(i + DEPTH, slot)

        return 0

    jax.lax.fori_loop(0, N_TILES, body, 0)
    for j in range(ODEPTH):
        i = N_TILES - ODEPTH + j
        out_copy(i, i % ODEPTH).wait()


def _finalize_kernel(cls_ref, attn_ref, mask_ref, bc_ref, ba_ref,
                     logits_ref, probs_ref, vp_ref, vl_ref, aw_ref):
    mask = mask_ref[...]
    logits = cls_ref[...] + bc_ref[0, 0]
    logits_ref[...] = logits
    probs = jax.nn.sigmoid(logits) * mask
    probs_ref[...] = probs

    k = TOPK_K

    def body(_, carry):
        lo, hi = carry
        mid = 0.5 * (lo + hi)
        cnt = jnp.sum((probs > mid).astype(jnp.float32), axis=1,
                      keepdims=True)
        below = cnt < float(k)
        hi = jnp.where(below, mid, hi)
        lo = jnp.where(below, lo, mid)
        return lo, hi

    lo0 = jnp.zeros((B_, 1), jnp.float32)
    hi0 = jnp.ones((B_, 1), jnp.float32)
    lo, hi = jax.lax.fori_loop(0, 30, body, (lo0, hi0))
    t = lo
    gt = probs > t
    cnt_gt = jnp.sum(gt.astype(jnp.float32), axis=1, keepdims=True)
    sum_gt = jnp.sum(jnp.where(gt, probs, 0.0), axis=1, keepdims=True)
    topk_score = (sum_gt + (float(k) - cnt_gt) * t) * (1.0 / float(k))

    alog = attn_ref[...] + ba_ref[0, 0]
    alog = jnp.where(mask == 0.0, -10000.0, alog)
    m = jnp.max(alog, axis=1, keepdims=True)
    e = jnp.exp(alog - m)
    s = jnp.sum(e, axis=1, keepdims=True)
    aw = e / s
    aw_ref[...] = aw
    attn_score = jnp.sum(aw * probs, axis=1, keepdims=True)

    video_prob = BETA * topk_score + (1.0 - BETA) * attn_score
    vp_ref[...] = video_prob
    p = jnp.clip(video_prob, 1e-6, 1.0 - 1e-6)
    vl_ref[...] = jnp.log(p / (1.0 - p))


def kernel(window_feat, window_mask, W_cls, b_cls, W_attn, b_attn):
    feat2d = window_feat.reshape(B_ * W_, DIM_)
    wcat = jnp.concatenate([W_cls, W_attn], axis=1)

    logits2 = pl.pallas_call(
        _matmul_kernel,
        in_specs=[
            pl.BlockSpec(memory_space=pltpu.MemorySpace.HBM),
            pl.BlockSpec((DIM_, 2), lambda: (0, 0)),
        ],
        out_specs=pl.BlockSpec(memory_space=pltpu.MemorySpace.HBM),
        out_shape=jax.ShapeDtypeStruct((B_ * W_, 2), jnp.float32),
        scratch_shapes=[
            pltpu.VMEM((DEPTH, TILE_A, DIM_), jnp.float32),
            pltpu.SemaphoreType.DMA((DEPTH,)),
            pltpu.VMEM((ODEPTH, TILE_A, 2), jnp.float32),
            pltpu.SemaphoreType.DMA((ODEPTH,)),
        ],
    )(feat2d, wcat)

    cls_l = logits2[:, 0].reshape(B_, W_)
    attn_l = logits2[:, 1].reshape(B_, W_)
    mask = window_mask.astype(jnp.float32)
    bc = b_cls.reshape(1, 1)
    ba = b_attn.reshape(1, 1)

    outs = pl.pallas_call(
        _finalize_kernel,
        out_shape=[
            jax.ShapeDtypeStruct((B_, W_), jnp.float32),
            jax.ShapeDtypeStruct((B_, W_), jnp.float32),
            jax.ShapeDtypeStruct((B_, 1), jnp.float32),
            jax.ShapeDtypeStruct((B_, 1), jnp.float32),
            jax.ShapeDtypeStruct((B_, W_), jnp.float32),
        ],
    )(cls_l, attn_l, mask, bc, ba)

    logits, probs, vp, vl, aw = outs
    return (logits, probs, vp.reshape(B_), vl.reshape(B_), aw)


# speedup vs baseline: 1.0010x; 1.0010x over previous
import jax
import jax.numpy as jnp
from jax.experimental import pallas as pl
from jax.experimental.pallas import tpu as pltpu

DIM_ = 1024
B_, W_ = 64, 2048
TOPK_K = max(1, int(round(W_ * 0.1)))
BETA = 0.6
TILE_A = 256
DEPTH = 40
N_TILES = (B_ * W_) // TILE_A


ODEPTH = 4


def _matmul_kernel(x_hbm, w_ref, o_hbm, buf, sem, obuf, osem):
    wbf = w_ref[...].astype(jnp.bfloat16)

    def start_copy(i, slot):
        pltpu.make_async_copy(
            x_hbm.at[pl.ds(i * TILE_A, TILE_A), :],
            buf.at[slot], sem.at[slot]).start()

    def out_copy(i, oslot):
        return pltpu.make_async_copy(
            obuf.at[oslot],
            o_hbm.at[pl.ds(i * TILE_A, TILE_A), :], osem.at[oslot])

    for j in range(DEPTH):
        start_copy(j, j)

    def body(i, _):
        slot = jax.lax.rem(i, DEPTH)
        oslot = jax.lax.rem(i, ODEPTH)
        pltpu.make_async_copy(
            x_hbm.at[pl.ds(i * TILE_A, TILE_A), :],
            buf.at[slot], sem.at[slot]).wait()

        @pl.when(i >= ODEPTH)
        def _():
            out_copy(i - ODEPTH, oslot).wait()

        obuf[oslot] = jnp.dot(
            buf[slot].astype(jnp.bfloat16), wbf,
            preferred_element_type=jnp.float32)
        out_copy(i, oslot).start()

        @pl.when(i + DEPTH < N_TILES)
        def _():
            start_copy(i + DEPTH, slot)

        return 0

    jax.lax.fori_loop(0, N_TILES, body, 0)
    for j in range(ODEPTH):
        i = N_TILES - ODEPTH + j
        out_copy(i, i % ODEPTH).wait()


def _finalize_kernel(cls_ref, attn_ref, mask_ref, bc_ref, ba_ref,
                     logits_ref, probs_ref, vp_ref, vl_ref, aw_ref):
    mask = mask_ref[...]
    logits = cls_ref[...] + bc_ref[0, 0]
    logits_ref[...] = logits
    probs = jax.nn.sigmoid(logits) * mask
    probs_ref[...] = probs

    k = TOPK_K

    def body(_, carry):
        lo, hi = carry
        mid = 0.5 * (lo + hi)
        cnt = jnp.sum((probs > mid).astype(jnp.float32), axis=1,
                      keepdims=True)
        below = cnt < float(k)
        hi = jnp.where(below, mid, hi)
        lo = jnp.where(below, lo, mid)
        return lo, hi

    lo0 = jnp.zeros((B_, 1), jnp.float32)
    hi0 = jnp.ones((B_, 1), jnp.float32)
    lo, hi = jax.lax.fori_loop(0, 30, body, (lo0, hi0))
    t = lo
    gt = probs > t
    cnt_gt = jnp.sum(gt.astype(jnp.float32), axis=1, keepdims=True)
    sum_gt = jnp.sum(jnp.where(gt, probs, 0.0), axis=1, keepdims=True)
    topk_score = (sum_gt + (float(k) - cnt_gt) * t) * (1.0 / float(k))

    alog = attn_ref[...] + ba_ref[0, 0]
    alog = jnp.where(mask == 0.0, -10000.0, alog)
    m = jnp.max(alog, axis=1, keepdims=True)
    e = jnp.exp(alog - m)
    s = jnp.sum(e, axis=1, keepdims=True)
    aw = e / s
    aw_ref[...] = aw
    attn_score = jnp.sum(aw * probs, axis=1, keepdims=True)

    video_prob = BETA * topk_score + (1.0 - BETA) * attn_score
    vp_ref[...] = video_prob
    p = jnp.clip(video_prob, 1e-6, 1.0 - 1e-6)
    vl_ref[...] = jnp.log(p / (1.0 - p))


def kernel(window_feat, window_mask, W_cls, b_cls, W_attn, b_attn):
    feat2d = window_feat.reshape(B_ * W_, DIM_)
    wcat = jnp.concatenate([W_cls, W_attn], axis=1)

    logits2 = pl.pallas_call(
        _matmul_kernel,
        in_specs=[
            pl.BlockSpec(memory_space=pltpu.MemorySpace.HBM),
            pl.BlockSpec((DIM_, 2), lambda: (0, 0)),
        ],
        out_specs=pl.BlockSpec(memory_space=pltpu.MemorySpace.HBM),
        out_shape=jax.ShapeDtypeStruct((B_ * W_, 2), jnp.float32),
        scratch_shapes=[
            pltpu.VMEM((DEPTH, TILE_A, DIM_), jnp.float32),
            pltpu.SemaphoreType.DMA((DEPTH,)),
            pltpu.VMEM((ODEPTH, TILE_A, 2), jnp.float32),
            pltpu.SemaphoreType.DMA((ODEPTH,)),
        ],
    )(feat2d, wcat)

    cls_l = logits2[:, 0].reshape(B_, W_)
    attn_l = logits2[:, 1].reshape(B_, W_)
    mask = window_mask.astype(jnp.float32)
    bc = b_cls.reshape(1, 1)
    ba = b_attn.reshape(1, 1)

    outs = pl.pallas_call(
        _finalize_kernel,
        out_shape=[
            jax.ShapeDtypeStruct((B_, W_), jnp.float32),
            jax.ShapeDtypeStruct((B_, W_), jnp.float32),
            jax.ShapeDtypeStruct((B_, 1), jnp.float32),
            jax.ShapeDtypeStruct((B_, 1), jnp.float32),
            jax.ShapeDtypeStruct((B_, W_), jnp.float32),
        ],
    )(cls_l, attn_l, mask, bc, ba)

    logits, probs, vp, vl, aw = outs
    return (logits, probs, vp.reshape(B_), vl.reshape(B_), aw)


# speedup vs baseline: 1.0015x; 1.0004x over previous
import jax
import jax.numpy as jnp
from jax.experimental import pallas as pl
from jax.experimental.pallas import tpu as pltpu

DIM_ = 1024
B_, W_ = 64, 2048
TOPK_K = max(1, int(round(W_ * 0.1)))
BETA = 0.6
TILE_A = 256
DEPTH = 24
N_TILES = (B_ * W_) // TILE_A


ODEPTH = 4


def _matmul_kernel(x_hbm, w_ref, o_hbm, buf, sem, obuf, osem):
    wbf = w_ref[...].astype(jnp.bfloat16)

    def start_copy(i, slot):
        pltpu.make_async_copy(
            x_hbm.at[pl.ds(i * TILE_A, TILE_A), :],
            buf.at[slot], sem.at[slot]).start()

    def out_copy(i, oslot):
        return pltpu.make_async_copy(
            obuf.at[oslot],
            o_hbm.at[pl.ds(i * TILE_A, TILE_A), :], osem.at[oslot])

    for j in range(DEPTH):
        start_copy(j, j)

    def body(i, _):
        slot = jax.lax.rem(i, DEPTH)
        oslot = jax.lax.rem(i, ODEPTH)
        pltpu.make_async_copy(
            x_hbm.at[pl.ds(i * TILE_A, TILE_A), :],
            buf.at[slot], sem.at[slot]).wait()

        @pl.when(i >= ODEPTH)
        def _():
            out_copy(i - ODEPTH, oslot).wait()

        obuf[oslot] = jnp.dot(
            buf[slot].astype(jnp.bfloat16), wbf,
            preferred_element_type=jnp.float32)
        out_copy(i, oslot).start()

        @pl.when(i + DEPTH < N_TILES)
        def _():
            start_copy(i + DEPTH, slot)

        return 0

    jax.lax.fori_loop(0, N_TILES, body, 0)
    for j in range(ODEPTH):
        i = N_TILES - ODEPTH + j
        out_copy(i, i % ODEPTH).wait()


def _finalize_kernel(cls_ref, attn_ref, mask_ref, bc_ref, ba_ref,
                     logits_ref, probs_ref, vp_ref, vl_ref, aw_ref):
    mask = mask_ref[...]
    logits = cls_ref[...] + bc_ref[0, 0]
    logits_ref[...] = logits
    probs = jax.nn.sigmoid(logits) * mask
    probs_ref[...] = probs

    k = TOPK_K

    def body(_, carry):
        lo, hi = carry
        mid = 0.5 * (lo + hi)
        cnt = jnp.sum((probs > mid).astype(jnp.float32), axis=1,
                      keepdims=True)
        below = cnt < float(k)
        hi = jnp.where(below, mid, hi)
        lo = jnp.where(below, lo, mid)
        return lo, hi

    lo0 = jnp.zeros((B_, 1), jnp.float32)
    hi0 = jnp.ones((B_, 1), jnp.float32)
    lo, hi = jax.lax.fori_loop(0, 30, body, (lo0, hi0))
    t = lo
    gt = probs > t
    cnt_gt = jnp.sum(gt.astype(jnp.float32), axis=1, keepdims=True)
    sum_gt = jnp.sum(jnp.where(gt, probs, 0.0), axis=1, keepdims=True)
    topk_score = (sum_gt + (float(k) - cnt_gt) * t) * (1.0 / float(k))

    alog = attn_ref[...] + ba_ref[0, 0]
    alog = jnp.where(mask == 0.0, -10000.0, alog)
    m = jnp.max(alog, axis=1, keepdims=True)
    e = jnp.exp(alog - m)
    s = jnp.sum(e, axis=1, keepdims=True)
    aw = e / s
    aw_ref[...] = aw
    attn_score = jnp.sum(aw * probs, axis=1, keepdims=True)

    video_prob = BETA * topk_score + (1.0 - BETA) * attn_score
    vp_ref[...] = video_prob
    p = jnp.clip(video_prob, 1e-6, 1.0 - 1e-6)
    vl_ref[...] = jnp.log(p / (1.0 - p))


def kernel(window_feat, window_mask, W_cls, b_cls, W_attn, b_attn):
    feat2d = window_feat.reshape(B_ * W_, DIM_)
    wcat = jnp.concatenate([W_cls, W_attn], axis=1)

    logits2 = pl.pallas_call(
        _matmul_kernel,
        in_specs=[
            pl.BlockSpec(memory_space=pltpu.MemorySpace.HBM),
            pl.BlockSpec((DIM_, 2), lambda: (0, 0)),
        ],
        out_specs=pl.BlockSpec(memory_space=pltpu.MemorySpace.HBM),
        out_shape=jax.ShapeDtypeStruct((B_ * W_, 2), jnp.float32),
        scratch_shapes=[
            pltpu.VMEM((DEPTH, TILE_A, DIM_), jnp.float32),
            pltpu.SemaphoreType.DMA((DEPTH,)),
            pltpu.VMEM((ODEPTH, TILE_A, 2), jnp.float32),
            pltpu.SemaphoreType.DMA((ODEPTH,)),
        ],
    )(feat2d, wcat)

    cls_l = logits2[:, 0].reshape(B_, W_)
    attn_l = logits2[:, 1].reshape(B_, W_)
    mask = window_mask.astype(jnp.float32)
    bc = b_cls.reshape(1, 1)
    ba = b_attn.reshape(1, 1)

    outs = pl.pallas_call(
        _finalize_kernel,
        out_shape=[
            jax.ShapeDtypeStruct((B_, W_), jnp.float32),
            jax.ShapeDtypeStruct((B_, W_), jnp.float32),
            jax.ShapeDtypeStruct((B_, 1), jnp.float32),
            jax.ShapeDtypeStruct((B_, 1), jnp.float32),
            jax.ShapeDtypeStruct((B_, W_), jnp.float32),
        ],
    )(cls_l, attn_l, mask, bc, ba)

    logits, probs, vp, vl, aw = outs
    return (logits, probs, vp.reshape(B_), vl.reshape(B_), aw)
